# baseline (speedup 1.0000x reference)
import jax
import jax.numpy as jnp
from jax.experimental import pallas as pl

_TOP_K = 8
_N_EXPERTS = 64
_IN_F = 1024
_OUT_F = 64
_N_TOKENS = 2048
_TOKB = 256


def _nested_fc_kernel(f_ref, a_ref, w_ref, b_ref, o_ref):
    f = f_ref[...]
    acts = a_ref[...]
    w = w_ref[...]
    allout = jnp.dot(f, w, preferred_element_type=jnp.float32)
    allout = allout.reshape(_TOKB, _OUT_F, _N_EXPERTS) + b_ref[...][None, :, :]

    eidx = jax.lax.broadcasted_iota(jnp.int32, (_TOKB, _N_EXPERTS), 1)
    cur = acts
    for k in range(_TOP_K):
        m = jnp.min(cur, axis=1, keepdims=True)
        sel = jnp.min(jnp.where(cur == m, eidx, _N_EXPERTS),
                      axis=1, keepdims=True)
        onehot = eidx == sel
        mask3 = onehot.reshape(_TOKB, 1, _N_EXPERTS)
        out_k = jnp.sum(jnp.where(mask3, allout, 0.0), axis=2)
        o_ref[:, k, :] = out_k
        cur = jnp.where(onehot, jnp.float32(jnp.inf), cur)


def kernel(features, activated, W, b):
    w2d = jnp.transpose(W, (1, 2, 0)).reshape(_IN_F, _OUT_F * _N_EXPERTS)
    bT = jnp.transpose(b, (1, 0))
    grid = (_N_TOKENS // _TOKB,)
    return pl.pallas_call(
        _nested_fc_kernel,
        grid=grid,
        in_specs=[
            pl.BlockSpec((_TOKB, _IN_F), lambda i: (i, 0)),
            pl.BlockSpec((_TOKB, _N_EXPERTS), lambda i: (i, 0)),
            pl.BlockSpec((_IN_F, _OUT_F * _N_EXPERTS), lambda i: (0, 0)),
            pl.BlockSpec((_OUT_F, _N_EXPERTS), lambda i: (0, 0)),
        ],
        out_specs=pl.BlockSpec((_TOKB, _TOP_K, _OUT_F), lambda i: (i, 0, 0)),
        out_shape=jax.ShapeDtypeStruct((_N_TOKENS, _TOP_K, _OUT_F), jnp.float32),
    )(features, activated, w2d, bT)


# speedup vs baseline: 1.5975x; 1.5975x over previous
import jax
import jax.numpy as jnp
from jax.experimental import pallas as pl

_TOP_K = 8
_N_EXPERTS = 64
_IN_F = 1024
_OUT_F = 64
_N_TOKENS = 2048
_TOKB = 256


def _nested_fc_kernel(f_ref, a_ref, w_ref, b_ref, o_ref):
    f = f_ref[...]
    acts = a_ref[...]
    w = w_ref[...]
    allout = jnp.dot(f, w, preferred_element_type=jnp.float32)
    allout = allout.reshape(_TOKB, _OUT_F, _N_EXPERTS) + b_ref[...][None, :, :]

    eidx = jax.lax.broadcasted_iota(jnp.int32, (_TOKB, _N_EXPERTS), 1)
    cur = acts
    sels = []
    for k in range(_TOP_K):
        m = jnp.min(cur, axis=1, keepdims=True)
        sel = jnp.min(jnp.where(cur == m, eidx, _N_EXPERTS),
                      axis=1, keepdims=True)
        sels.append(sel)
        cur = jnp.where(eidx == sel, jnp.float32(jnp.inf), cur)
    selk = jnp.concatenate(sels, axis=1)
    idx3 = jnp.broadcast_to(selk.reshape(_TOKB, 1, _TOP_K),
                            (_TOKB, _OUT_F, _TOP_K))
    o_ref[...] = jnp.take_along_axis(allout, idx3, axis=2)


def kernel(features, activated, W, b):
    w2d = jnp.transpose(W, (1, 2, 0)).reshape(_IN_F, _OUT_F * _N_EXPERTS)
    bT = jnp.transpose(b, (1, 0))
    grid = (_N_TOKENS // _TOKB,)
    out = pl.pallas_call(
        _nested_fc_kernel,
        grid=grid,
        in_specs=[
            pl.BlockSpec((_TOKB, _IN_F), lambda i: (i, 0)),
            pl.BlockSpec((_TOKB, _N_EXPERTS), lambda i: (i, 0)),
            pl.BlockSpec((_IN_F, _OUT_F * _N_EXPERTS), lambda i: (0, 0)),
            pl.BlockSpec((_OUT_F, _N_EXPERTS), lambda i: (0, 0)),
        ],
        out_specs=pl.BlockSpec((_TOKB, _OUT_F, _TOP_K), lambda i: (i, 0, 0)),
        out_shape=jax.ShapeDtypeStruct((_N_TOKENS, _OUT_F, _TOP_K), jnp.float32),
    )(features, activated, w2d, bT)
    return jnp.transpose(out, (0, 2, 1))


# speedup vs baseline: 1.7231x; 1.0787x over previous
import jax
import jax.numpy as jnp
from jax.experimental import pallas as pl

_TOP_K = 8
_N_EXPERTS = 64
_IN_F = 1024
_OUT_F = 64
_N_TOKENS = 2048
_TOKB = 256


def _nested_fc_kernel(f_ref, a_ref, w_ref, b_ref, o_ref):
    f = f_ref[...]
    acts = a_ref[...]
    w = w_ref[...]
    allout = jnp.dot(f, w, preferred_element_type=jnp.float32)
    allout = allout + b_ref[...]
    allout = allout.reshape(_TOKB, _OUT_F, _N_EXPERTS)

    eidx = jax.lax.broadcasted_iota(jnp.int32, (_TOKB, _N_EXPERTS), 1)
    cur = acts
    sels = []
    for k in range(_TOP_K):
        m = jnp.min(cur, axis=1, keepdims=True)
        sel = jnp.min(jnp.where(cur == m, eidx, _N_EXPERTS),
                      axis=1, keepdims=True)
        sels.append(sel)
        cur = jnp.where(eidx == sel, jnp.float32(jnp.inf), cur)
    selk = jnp.concatenate(sels, axis=1)
    idx3 = jnp.broadcast_to(selk.reshape(_TOKB, 1, _TOP_K),
                            (_TOKB, _OUT_F, _TOP_K))
    o_ref[...] = jnp.take_along_axis(allout, idx3, axis=2)


def kernel(features, activated, W, b):
    w2d = jnp.transpose(W, (1, 2, 0)).reshape(
        _IN_F, _OUT_F * _N_EXPERTS).astype(jnp.bfloat16)
    b2d = jnp.transpose(b, (1, 0)).reshape(1, _OUT_F * _N_EXPERTS)
    f16 = features.astype(jnp.bfloat16)
    grid = (_N_TOKENS // _TOKB,)
    out = pl.pallas_call(
        _nested_fc_kernel,
        grid=grid,
        in_specs=[
            pl.BlockSpec((_TOKB, _IN_F), lambda i: (i, 0)),
            pl.BlockSpec((_TOKB, _N_EXPERTS), lambda i: (i, 0)),
            pl.BlockSpec((_IN_F, _OUT_F * _N_EXPERTS), lambda i: (0, 0)),
            pl.BlockSpec((1, _OUT_F * _N_EXPERTS), lambda i: (0, 0)),
        ],
        out_specs=pl.BlockSpec((_TOKB, _OUT_F, _TOP_K), lambda i: (i, 0, 0)),
        out_shape=jax.ShapeDtypeStruct((_N_TOKENS, _OUT_F, _TOP_K), jnp.float32),
    )(f16, activated, w2d, b2d)
    return jnp.transpose(out, (0, 2, 1))
